# Initial kernel scaffold; baseline (speedup 1.0000x reference)
import jax
import jax.numpy as jnp
from jax.experimental import pallas as pl


def kernel(inputs, targets):
    raise NotImplementedError("write your pallas kernel here")


# speedup vs baseline: 7.3353x; 7.3353x over previous
import functools

import jax
import jax.numpy as jnp
from jax import lax
from jax.experimental import pallas as pl
from jax.experimental.pallas import tpu as pltpu
from jax.experimental.pallas import tpu_sc as plsc

NCLS = 10000
BATCH = 4096
DIM = 512
MIX_ALPHA = 0.2

NWORKERS = 32
ROWS_PER_W = BATCH // NWORKERS
CHUNK = 8
NCHUNKS = ROWS_PER_W // CHUNK
LANES = 16


TC_BLK = 128


def _tc_mix_body(lam_ref, p_ref, a_ref, b_ref, o_ref):
    lam = lam_ref[0, 0]
    rev = jnp.dot(p_ref[...], b_ref[...], preferred_element_type=jnp.float32)
    o_ref[...] = a_ref[...] * lam + rev * (1.0 - lam)


_sc_mesh = plsc.VectorSubcoreMesh(core_axis_name="c", subcore_axis_name="s")


@functools.partial(
    pl.kernel,
    mesh=_sc_mesh,
    compiler_params=pltpu.CompilerParams(needs_layout_passes=False),
    out_type=jax.ShapeDtypeStruct((BATCH, NCLS), jnp.float32),
    scratch_types=[
        pltpu.VMEM((ROWS_PER_W,), jnp.int32),
        pltpu.VMEM((ROWS_PER_W,), jnp.int32),
        pltpu.VMEM((2 * LANES,), jnp.float32),
        pltpu.VMEM((CHUNK, NCLS), jnp.float32),
    ],
)
def _sc_targets(tgt_hbm, vals_hbm, out_hbm, tgt_v, rev_v, vals_v, buf):
    cid = lax.axis_index("c")
    sid = lax.axis_index("s")
    wid = sid * 2 + cid
    base = wid * ROWS_PER_W

    pltpu.sync_copy(tgt_hbm.at[pl.ds(base, ROWS_PER_W)], tgt_v)
    pltpu.sync_copy(
        tgt_hbm.at[pl.ds(BATCH - base - ROWS_PER_W, ROWS_PER_W)], rev_v)
    pltpu.sync_copy(vals_hbm, vals_v)

    zf = jnp.zeros((LANES,), jnp.float32)
    for r in range(CHUNK):
        def _zero_body(i, _, r=r):
            buf[r, pl.ds(i * LANES, LANES)] = zf
            return 0
        lax.fori_loop(0, NCLS // LANES, _zero_body, 0)

    jlane = lax.iota(jnp.int32, 16)
    jloc = jlane & 7
    lo = jlane < 8
    v_nocoll = vals_v[pl.ds(0, LANES)]
    v_coll = vals_v[pl.ds(LANES, LANES)]

    for chunk in range(NCHUNKS):
        idx_l = chunk * CHUNK + jloc
        ca = plsc.load_gather(tgt_v, [idx_l])
        cb = plsc.load_gather(rev_v, [(ROWS_PER_W - 1) - idx_l])
        coll = ca == cb
        cols = jnp.where(lo, ca, cb)
        vals = jnp.where(coll, v_coll, v_nocoll)
        mask = jnp.logical_or(lo, jnp.logical_not(coll))
        plsc.store_scatter(buf, [jloc, cols], vals, mask=mask)
        pltpu.sync_copy(buf, out_hbm.at[pl.ds(base + chunk * CHUNK, CHUNK)])
        plsc.store_scatter(buf, [jloc, cols], zf, mask=mask)


def kernel(inputs, targets):
    lam = jax.random.beta(jax.random.key(42), MIX_ALPHA, MIX_ALPHA)
    lam = lam.astype(jnp.float32)
    lamc = 1.0 - lam

    nblk = BATCH // TC_BLK
    perm = jnp.flipud(jnp.eye(TC_BLK, dtype=jnp.float32))
    inputs_mixed = pl.pallas_call(
        _tc_mix_body,
        grid=(nblk,),
        in_specs=[
            pl.BlockSpec((1, 1), lambda i: (0, 0)),
            pl.BlockSpec((TC_BLK, TC_BLK), lambda i: (0, 0)),
            pl.BlockSpec((TC_BLK, DIM), lambda i: (i, 0)),
            pl.BlockSpec((TC_BLK, DIM), lambda i: (nblk - 1 - i, 0)),
        ],
        out_specs=pl.BlockSpec((TC_BLK, DIM), lambda i: (i, 0)),
        out_shape=jax.ShapeDtypeStruct((BATCH, DIM), jnp.float32),
    )(lam.reshape(1, 1), perm, inputs, inputs)

    vals = jnp.concatenate([
        jnp.full((8,), lam, jnp.float32),
        jnp.full((8,), lamc, jnp.float32),
        jnp.full((LANES,), lam + lamc, jnp.float32),
    ])
    targets_mixed = _sc_targets(targets, vals)

    return (inputs_mixed, targets_mixed)
